# baseline (speedup 1.0000x reference)
import functools

import jax
import jax.numpy as jnp
from jax import lax
from jax.experimental import pallas as pl
from jax.experimental.pallas import tpu as pltpu
from jax.experimental.pallas import tpu_sc as plsc

B = 1024
N = 100000
H = 200
HP = 208
K = 20
L = 16
C = 20000
NC = N // C
U = 10
STEP = L * U
ITERS = C // STEP
CAP = 1024
TRIG = 512
NEG_INF = float("-inf")
BIGI = 2 ** 30

_info = plsc.get_sparse_core_info()
NCORES = _info.num_cores
NSUB = _info.num_subcores
NW = NCORES * NSUB
ROWS_PW = B // NW
TOTAL_CHUNKS = ROWS_PW * NC


def _body(scores_hbm, hist_hbm, outv_hbm, outi_hbm,
          chunks, histv, bufv, bufi, stagev, stagei, sem):
    wid = lax.axis_index("s") * NCORES + lax.axis_index("c")
    region = wid * (ROWS_PW * N)
    iota = lax.iota(jnp.int32, L)
    ninf_v = jnp.full((L,), NEG_INF, jnp.float32)
    bigi_v = jnp.full((L,), BIGI, jnp.int32)

    def start_chunk(flat):
        off = region + flat * C
        par = lax.rem(flat, 2)
        pltpu.make_async_copy(scores_hbm.at[pl.ds(off, C)],
                              chunks.at[pl.ds(par * C, C)], sem).start()

    def wait_chunk(flat, par):
        off = region + flat * C
        pltpu.make_async_copy(scores_hbm.at[pl.ds(off, C)],
                              chunks.at[pl.ds(par * C, C)], sem).wait()

    def select_topk(cnt):
        nv = (cnt + (L - 1)) // L

        def ext_body(t, carry):
            vlo, vhi, ilo, ihi, _ = carry

            def scan_vreg(i, bc):
                bv, bi = bc
                pos = i * L + iota
                v = bufv[pl.ds(i * L, L)]
                ci = bufi[pl.ds(i * L, L)]
                valid = pos < cnt
                v = jnp.where(valid, v, ninf_v)
                ci = jnp.where(valid, ci, bigi_v)
                better = (v > bv) | ((v == bv) & (ci < bi))
                return jnp.where(better, v, bv), jnp.where(better, ci, bi)

            bv, bi = lax.fori_loop(0, nv, scan_vreg, (ninf_v, bigi_v))
            mv = jnp.max(bv)
            mi = jnp.min(jnp.where(bv == mv, bi, bigi_v))

            def rm_body(i, tok):
                v = bufv[pl.ds(i * L, L)]
                ci = bufi[pl.ds(i * L, L)]
                sel = (v == mv) & (ci == mi)
                bufv[pl.ds(i * L, L)] = jnp.where(sel, ninf_v, v)
                return tok

            lax.fori_loop(0, nv, rm_body, jnp.int32(0))

            mvv = jnp.full((L,), mv)
            miv = jnp.full((L,), mi)
            lsel = iota == lax.rem(t, L)
            lo = t < L
            vlo = jnp.where(lsel & lo, mvv, vlo)
            ilo = jnp.where(lsel & lo, miv, ilo)
            vhi = jnp.where(lsel & jnp.logical_not(lo), mvv, vhi)
            ihi = jnp.where(lsel & jnp.logical_not(lo), miv, ihi)
            return vlo, vhi, ilo, ihi, mv

        init = (ninf_v, ninf_v, bigi_v, bigi_v, jnp.float32(NEG_INF))
        return lax.fori_loop(0, K, ext_body, init)

    def compact(cnt):
        vlo, vhi, ilo, ihi, kth = select_topk(cnt)
        bufv[pl.ds(0, L)] = vlo
        bufv[pl.ds(L, L)] = vhi
        bufi[pl.ds(0, L)] = ilo
        bufi[pl.ds(L, L)] = ihi
        return jnp.full((L,), kth), kth, jnp.int32(2 * L)

    def hist_scatter(cbase, par):
        for h in range(HP // L):
            hv = histv[pl.ds(h * L, L)]
            col = hv - 1
            m = (hv > 0) & (col >= cbase) & (col < cbase + C)
            lidx = jnp.where(m, col - cbase + par * C, par * C)
            plsc.store_scatter(chunks, [lidx], ninf_v, mask=m)

    def make_scan_step(cbase, par):
        def scan_step(i, tc):
            thresh, thresh_s, cnt = tc
            base = par * C + i * STEP
            vs = [chunks[pl.ds(base + j * L, L)] for j in range(U)]
            mx = list(vs)
            while len(mx) > 1:
                mx = [jnp.maximum(mx[t], mx[t + 1])
                      for t in range(0, len(mx) - 1, 2)] + (
                          [mx[-1]] if len(mx) % 2 else [])
            mx_s = jnp.max(mx[0])

            def do_insert(tc):
                thresh, thresh_s, cnt = tc
                cntv = jnp.full((L,), cnt, jnp.int32)
                for j in range(U):
                    v = chunks[pl.ds(base + j * L, L)]
                    m = v >= thresh
                    mi32 = m.astype(jnp.int32)
                    col = (cbase + i * STEP + j * L) + iota
                    pos = cntv + jnp.cumsum(mi32) - mi32
                    plsc.store_scatter(bufv, [pos], v, mask=m)
                    plsc.store_scatter(bufi, [pos], col, mask=m)
                    cntv = cntv + plsc.all_reduce_population_count(m)
                cnt = jnp.max(cntv)
                return lax.cond(cnt > TRIG, compact,
                                lambda c, t=thresh, ts=thresh_s: (t, ts, c),
                                cnt)

            return lax.cond(mx_s >= thresh_s, do_insert, lambda tc: tc,
                            (thresh, thresh_s, cnt))

        return scan_step

    def row_body(r, tok):
        rowg = wid * ROWS_PW + r
        pltpu.sync_copy(hist_hbm.at[pl.ds(rowg * HP, HP)], histv)

        def make_warm_step(par):
            def warm_step(i, carry):
                t0, t1, t2 = carry
                base = par * C + i * STEP
                for j in range(U):
                    v = chunks[pl.ds(base + j * L, L)]
                    lo0 = jnp.minimum(t0, v)
                    t0 = jnp.maximum(t0, v)
                    lo1 = jnp.minimum(t1, lo0)
                    t1 = jnp.maximum(t1, lo0)
                    t2 = jnp.maximum(t2, lo1)
                return t0, t1, t2
            return warm_step

        def thr_step(t, carry):
            t0, t1, t2, _ = carry
            mx = jnp.max(jnp.maximum(jnp.maximum(t0, t1), t2))
            t0 = jnp.where(t0 == mx, ninf_v, t0)
            t1 = jnp.where(t1 == mx, ninf_v, t1)
            t2 = jnp.where(t2 == mx, ninf_v, t2)
            return t0, t1, t2, mx

        tc = None
        for c in range(NC):
            flat = r * NC + c
            par = lax.rem(flat, 2)
            wait_chunk(flat, par)

            @pl.when(flat + 1 < TOTAL_CHUNKS)
            def _():
                start_chunk(flat + 1)

            hist_scatter(c * C, par)
            if c == 0:
                t0, t1, t2 = lax.fori_loop(0, ITERS, make_warm_step(par),
                                           (ninf_v, ninf_v, ninf_v))
                _, _, _, th20 = lax.fori_loop(
                    0, K, thr_step, (t0, t1, t2, jnp.float32(NEG_INF)))
                tc = (jnp.full((L,), th20), th20, jnp.int32(0))
            tc = lax.fori_loop(0, ITERS, make_scan_step(c * C, par), tc)
        cnt = tc[2]
        vlo, vhi, ilo, ihi, _ = select_topk(cnt)
        stagev[pl.ds(r * 32, L)] = vlo
        stagev[pl.ds(r * 32 + L, L)] = vhi
        stagei[pl.ds(r * 32, L)] = ilo + 1
        stagei[pl.ds(r * 32 + L, L)] = ihi + 1
        return tok

    start_chunk(jnp.int32(0))
    lax.fori_loop(0, ROWS_PW, row_body, jnp.int32(0))
    pltpu.sync_copy(stagev, outv_hbm.at[pl.ds(wid * (ROWS_PW * 32),
                                              ROWS_PW * 32)])
    pltpu.sync_copy(stagei, outi_hbm.at[pl.ds(wid * (ROWS_PW * 32),
                                              ROWS_PW * 32)])


@jax.jit
def _sc_topk(scores_flat, hist_flat):
    mesh = plsc.VectorSubcoreMesh(core_axis_name="c", subcore_axis_name="s")
    f = functools.partial(
        pl.kernel,
        mesh=mesh,
        compiler_params=pltpu.CompilerParams(needs_layout_passes=False),
        out_type=[jax.ShapeDtypeStruct((B * 32,), jnp.float32),
                  jax.ShapeDtypeStruct((B * 32,), jnp.int32)],
        scratch_types=[
            pltpu.VMEM((2 * C,), jnp.float32),
            pltpu.VMEM((HP,), jnp.int32),
            pltpu.VMEM((CAP,), jnp.float32),
            pltpu.VMEM((CAP,), jnp.int32),
            pltpu.VMEM((ROWS_PW * 32,), jnp.float32),
            pltpu.VMEM((ROWS_PW * 32,), jnp.int32),
            pltpu.SemaphoreType.DMA,
        ],
    )(_body)
    return f(scores_flat, hist_flat)


def kernel(scores, history, k):
    del k
    hist_p = jnp.pad(history, ((0, 0), (0, HP - H))).reshape(-1)
    vals32, items32 = _sc_topk(scores.reshape(-1), hist_p)
    vals = vals32.reshape(B, 32)[:, :K]
    items = items32.reshape(B, 32)[:, :K]
    return vals, items


# speedup vs baseline: 1.3083x; 1.3083x over previous
import functools

import jax
import jax.numpy as jnp
from jax import lax
from jax.experimental import pallas as pl
from jax.experimental.pallas import tpu as pltpu
from jax.experimental.pallas import tpu_sc as plsc

B = 1024
N = 100000
H = 200
HP = 208
K = 20
L = 16
C = 20000
NC = N // C
U = 10
STEP = L * U
ITERS = C // STEP
CAP = 256
NEG_INF = float("-inf")
BIGI = 2 ** 30

_info = plsc.get_sparse_core_info()
NCORES = _info.num_cores
NSUB = _info.num_subcores
NW = NCORES * NSUB
ROWS_PW = B // NW
TOTAL_CHUNKS = ROWS_PW * NC


def _body(scores_hbm, hist_hbm, outv_hbm, outi_hbm,
          chunks, histv, bufv, bufi, stagev, stagei, sem):
    wid = lax.axis_index("s") * NCORES + lax.axis_index("c")
    region = wid * (ROWS_PW * N)
    iota = lax.iota(jnp.int32, L)
    ninf_v = jnp.full((L,), NEG_INF, jnp.float32)
    bigi_v = jnp.full((L,), BIGI, jnp.int32)

    def start_chunk(flat):
        off = region + flat * C
        par = lax.rem(flat, 2)
        pltpu.make_async_copy(scores_hbm.at[pl.ds(off, C)],
                              chunks.at[pl.ds(par * C, C)], sem).start()

    def wait_chunk(flat, par):
        off = region + flat * C
        pltpu.make_async_copy(scores_hbm.at[pl.ds(off, C)],
                              chunks.at[pl.ds(par * C, C)], sem).wait()

    def select_topk(cnt):
        nv = (cnt + (L - 1)) // L

        def ext_body(t, carry):
            vlo, vhi, ilo, ihi, _ = carry

            def scan_vreg(i, bc):
                bv, bi = bc
                pos = i * L + iota
                v = bufv[pl.ds(i * L, L)]
                ci = bufi[pl.ds(i * L, L)]
                valid = pos < cnt
                v = jnp.where(valid, v, ninf_v)
                ci = jnp.where(valid, ci, bigi_v)
                better = (v > bv) | ((v == bv) & (ci < bi))
                return jnp.where(better, v, bv), jnp.where(better, ci, bi)

            bv, bi = lax.fori_loop(0, nv, scan_vreg, (ninf_v, bigi_v))
            mv = jnp.max(bv)
            mi = jnp.min(jnp.where(bv == mv, bi, bigi_v))

            def rm_body(i, tok):
                v = bufv[pl.ds(i * L, L)]
                ci = bufi[pl.ds(i * L, L)]
                sel = (v == mv) & (ci == mi)
                bufv[pl.ds(i * L, L)] = jnp.where(sel, ninf_v, v)
                return tok

            lax.fori_loop(0, nv, rm_body, jnp.int32(0))

            mvv = jnp.full((L,), mv)
            miv = jnp.full((L,), mi)
            lsel = iota == lax.rem(t, L)
            lo = t < L
            vlo = jnp.where(lsel & lo, mvv, vlo)
            ilo = jnp.where(lsel & lo, miv, ilo)
            vhi = jnp.where(lsel & jnp.logical_not(lo), mvv, vhi)
            ihi = jnp.where(lsel & jnp.logical_not(lo), miv, ihi)
            return vlo, vhi, ilo, ihi, mv

        init = (ninf_v, ninf_v, bigi_v, bigi_v, jnp.float32(NEG_INF))
        return lax.fori_loop(0, K, ext_body, init)

    def compact(cnt):
        vlo, vhi, ilo, ihi, kth = select_topk(cnt)
        bufv[pl.ds(0, L)] = vlo
        bufv[pl.ds(L, L)] = vhi
        bufi[pl.ds(0, L)] = ilo
        bufi[pl.ds(L, L)] = ihi
        return jnp.full((L,), kth), kth, jnp.int32(2 * L)

    def hist_scatter(cbase, par):
        for h in range(HP // L):
            hv = histv[pl.ds(h * L, L)]
            col = hv - 1
            m = (hv > 0) & (col >= cbase) & (col < cbase + C)
            lidx = jnp.where(m, col - cbase + par * C, par * C)
            plsc.store_scatter(chunks, [lidx], ninf_v, mask=m)

    def make_scan_step(cbase, par):
        def scan_step(i, tc):
            thresh, thresh_s, cnt = tc
            base = par * C + i * STEP
            vs = [chunks[pl.ds(base + j * L, L)] for j in range(U)]
            mx = list(vs)
            while len(mx) > 1:
                mx = [jnp.maximum(mx[t], mx[t + 1])
                      for t in range(0, len(mx) - 1, 2)] + (
                          [mx[-1]] if len(mx) % 2 else [])
            mx_s = jnp.max(mx[0])

            def do_insert(tc):
                thresh, thresh_s, cnt = tc
                for j in range(U):
                    v = chunks[pl.ds(base + j * L, L)]
                    m = v >= thresh
                    mi32 = m.astype(jnp.int32)
                    col = (cbase + i * STEP + j * L) + iota
                    pos = cnt + jnp.cumsum(mi32) - mi32
                    plsc.store_scatter(bufv, [pos], v, mask=m)
                    plsc.store_scatter(bufi, [pos], col, mask=m)
                    cnt = cnt + jnp.sum(mi32)
                    thresh, thresh_s, cnt = lax.cond(
                        cnt > CAP - L, compact,
                        lambda c, t=thresh, ts=thresh_s: (t, ts, c), cnt)
                return thresh, thresh_s, cnt

            return lax.cond(mx_s >= thresh_s, do_insert, lambda tc: tc,
                            (thresh, thresh_s, cnt))

        return scan_step

    def row_body(r, tok):
        rowg = wid * ROWS_PW + r
        pltpu.sync_copy(hist_hbm.at[pl.ds(rowg * HP, HP)], histv)

        def chunk_body(c, tc):
            flat = r * NC + c
            par = lax.rem(flat, 2)
            wait_chunk(flat, par)

            @pl.when(flat + 1 < TOTAL_CHUNKS)
            def _():
                start_chunk(flat + 1)

            cbase = c * C
            hist_scatter(cbase, par)
            return lax.fori_loop(0, ITERS, make_scan_step(cbase, par), tc)

        _, _, cnt = lax.fori_loop(0, NC, chunk_body,
                                  (jnp.full((L,), NEG_INF, jnp.float32),
                                   jnp.float32(NEG_INF), jnp.int32(0)))
        vlo, vhi, ilo, ihi, _ = select_topk(cnt)
        stagev[pl.ds(r * 32, L)] = vlo
        stagev[pl.ds(r * 32 + L, L)] = vhi
        stagei[pl.ds(r * 32, L)] = ilo + 1
        stagei[pl.ds(r * 32 + L, L)] = ihi + 1
        return tok

    start_chunk(jnp.int32(0))
    lax.fori_loop(0, ROWS_PW, row_body, jnp.int32(0))
    pltpu.sync_copy(stagev, outv_hbm.at[pl.ds(wid * (ROWS_PW * 32),
                                              ROWS_PW * 32)])
    pltpu.sync_copy(stagei, outi_hbm.at[pl.ds(wid * (ROWS_PW * 32),
                                              ROWS_PW * 32)])


@jax.jit
def _sc_topk(scores_flat, hist_flat):
    mesh = plsc.VectorSubcoreMesh(core_axis_name="c", subcore_axis_name="s")
    f = functools.partial(
        pl.kernel,
        mesh=mesh,
        compiler_params=pltpu.CompilerParams(needs_layout_passes=False),
        out_type=[jax.ShapeDtypeStruct((B * 32,), jnp.float32),
                  jax.ShapeDtypeStruct((B * 32,), jnp.int32)],
        scratch_types=[
            pltpu.VMEM((2 * C,), jnp.float32),
            pltpu.VMEM((HP,), jnp.int32),
            pltpu.VMEM((CAP,), jnp.float32),
            pltpu.VMEM((CAP,), jnp.int32),
            pltpu.VMEM((ROWS_PW * 32,), jnp.float32),
            pltpu.VMEM((ROWS_PW * 32,), jnp.int32),
            pltpu.SemaphoreType.DMA,
        ],
    )(_body)
    return f(scores_flat, hist_flat)


def kernel(scores, history, k):
    del k
    hist_p = jnp.pad(history, ((0, 0), (0, HP - H))).reshape(-1)
    vals32, items32 = _sc_topk(scores.reshape(-1), hist_p)
    vals = vals32.reshape(B, 32)[:, :K]
    items = items32.reshape(B, 32)[:, :K]
    return vals, items
